# baseline (speedup 1.0000x reference)
import functools

import jax
import jax.numpy as jnp
from jax.experimental import pallas as pl
from jax.experimental.pallas import tpu as pltpu

N = 10000
DIN = 128
DOUT = 128
HALF = N // 2
BMR = 200
NB = HALF // BMR
R = 3


def _manual_kernel(adj_ref, x_ref, w_own_ref, w_nbr_ref, bias_ref, out_ref,
                   buf_ref, h_ref, sems):
    def start(b):
        slot = jax.lax.rem(b, R)
        for s in range(2):
            pltpu.make_async_copy(
                adj_ref.at[pl.ds(s * HALF + b * BMR, BMR), :],
                buf_ref.at[slot, s],
                sems.at[slot, s],
            ).start()

    for r in range(R):
        start(r)

    h_ref[...] = jnp.dot(x_ref[...], w_nbr_ref[...],
                         preferred_element_type=jnp.float32
                         ).astype(jnp.bfloat16)

    def loop_body(b, carry):
        slot = jax.lax.rem(b, R)
        for s in range(2):
            pltpu.make_async_copy(
                adj_ref.at[pl.ds(s * HALF + b * BMR, BMR), :],
                buf_ref.at[slot, s],
                sems.at[slot, s],
            ).wait()
        h = h_ref[...]
        w_own = w_own_ref[...]
        bval = bias_ref[...]
        for s in range(2):
            blk = buf_ref[slot, s].astype(jnp.bfloat16)
            nbr = jnp.dot(blk, h, preferred_element_type=jnp.float32)
            own = jnp.dot(x_ref[pl.ds(s * HALF + b * BMR, BMR), :], w_own,
                          preferred_element_type=jnp.float32)
            out_ref[pl.ds(s * HALF + b * BMR, BMR), :] = nbr + own + bval

        @pl.when(b + R < NB)
        def _():
            start(b + R)

        return carry

    jax.lax.fori_loop(0, NB, loop_body, 0)


@functools.partial(jax.jit, static_argnames=())
def kernel(x, adj, weight_own, weight_nbr, bias):
    bias2d = bias.reshape(1, DOUT)
    out = pl.pallas_call(
        _manual_kernel,
        in_specs=[
            pl.BlockSpec(memory_space=pltpu.MemorySpace.HBM),
            pl.BlockSpec(memory_space=pltpu.MemorySpace.VMEM),
            pl.BlockSpec(memory_space=pltpu.MemorySpace.VMEM),
            pl.BlockSpec(memory_space=pltpu.MemorySpace.VMEM),
            pl.BlockSpec(memory_space=pltpu.MemorySpace.VMEM),
        ],
        out_specs=pl.BlockSpec(memory_space=pltpu.MemorySpace.VMEM),
        out_shape=jax.ShapeDtypeStruct((N, DOUT), jnp.float32),
        scratch_shapes=[
            pltpu.VMEM((R, 2, BMR, N), jnp.float32),
            pltpu.VMEM((N, DOUT), jnp.bfloat16),
            pltpu.SemaphoreType.DMA((R, 2)),
        ],
        compiler_params=pltpu.CompilerParams(
            vmem_limit_bytes=64 * 1024 * 1024,
        ),
    )(adj, x, weight_own, weight_nbr, bias2d)
    return out


# speedup vs baseline: 1.0410x; 1.0410x over previous
import functools

import jax
import jax.numpy as jnp
from jax.experimental import pallas as pl
from jax.experimental.pallas import tpu as pltpu

N = 10000
DIN = 128
DOUT = 128
BM = 200
HALF = N // 2


def _main_kernel(adj_t_ref, adj_b_ref, x_ref, w_own_ref, w_nbr_ref, bias_ref,
                 out_ref, h_ref):
    i = pl.program_id(0)

    @pl.when(i == 0)
    def _():
        h_ref[...] = jnp.dot(x_ref[...], w_nbr_ref[...],
                             preferred_element_type=jnp.float32
                             ).astype(jnp.bfloat16)

    h = h_ref[...]
    w_own = w_own_ref[...]
    b = bias_ref[...]
    x_t = x_ref[pl.ds(i * BM, BM), :]
    x_b = x_ref[pl.ds(HALF + i * BM, BM), :]
    top = jnp.dot(adj_t_ref[0].astype(jnp.bfloat16), h,
                  preferred_element_type=jnp.float32)
    bot = jnp.dot(adj_b_ref[0].astype(jnp.bfloat16), h,
                  preferred_element_type=jnp.float32)
    out_ref[0] = top + jnp.dot(x_t, w_own, preferred_element_type=jnp.float32) + b
    out_ref[1] = bot + jnp.dot(x_b, w_own, preferred_element_type=jnp.float32) + b


@functools.partial(jax.jit, static_argnames=())
def kernel(x, adj, weight_own, weight_nbr, bias):
    adj3 = adj.reshape(2, HALF, N)
    bias2d = bias.reshape(1, DOUT)
    grid = (HALF // BM,)
    out = pl.pallas_call(
        _main_kernel,
        grid=grid,
        in_specs=[
            pl.BlockSpec((1, BM, N), lambda i: (0, i, 0)),
            pl.BlockSpec((1, BM, N), lambda i: (1, i, 0)),
            pl.BlockSpec((N, DIN), lambda i: (0, 0)),
            pl.BlockSpec((DIN, DOUT), lambda i: (0, 0)),
            pl.BlockSpec((DIN, DOUT), lambda i: (0, 0)),
            pl.BlockSpec((1, DOUT), lambda i: (0, 0)),
        ],
        out_specs=pl.BlockSpec((2, BM, DOUT), lambda i: (0, i, 0)),
        out_shape=jax.ShapeDtypeStruct((2, HALF, DOUT), jnp.float32),
        scratch_shapes=[pltpu.VMEM((N, DOUT), jnp.bfloat16)],
        compiler_params=pltpu.CompilerParams(
            dimension_semantics=("arbitrary",),
        ),
    )(adj3, adj3, x, weight_own, weight_nbr, bias2d)
    return out.reshape(N, DOUT)
